# baseline (speedup 1.0000x reference)
import jax
import jax.numpy as jnp
from jax.experimental import pallas as pl

_N = 20000
_BN = 400


def _gcn_h(x, e, W, b, n):
    src = e[0]
    dst = e[1]
    loop = jnp.arange(n, dtype=src.dtype)
    src = jnp.concatenate([src, loop])
    dst = jnp.concatenate([dst, loop])
    deg = jnp.zeros((n,), x.dtype).at[dst].add(1.0)
    dinv = jax.lax.rsqrt(jnp.maximum(deg, 1.0))
    h = x @ W
    msg = h[src] * (dinv[src] * dinv[dst])[:, None]
    out = jnp.zeros((n, W.shape[1]), x.dtype).at[dst].add(msg)
    return out + b


def _sage_h(x_src, x_dst, e, Wl, bl, Wr, n_dst):
    src = e[0]
    dst = e[1]
    s = jnp.zeros((n_dst, x_src.shape[1]), x_src.dtype).at[dst].add(x_src[src])
    cnt = jnp.zeros((n_dst,), x_src.dtype).at[dst].add(1.0)
    mean = s / jnp.maximum(cnt, 1.0)[:, None]
    return mean @ Wl + bl + x_dst @ Wr


def _lr(x):
    return jax.nn.leaky_relu(x, 0.2)


def _branch_h(xl, xp, e_lp, e_ll, e_pp, p):
    sp = _sage_h(xl, xp, e_lp, p[0], p[1], p[2], _N)
    gl = _gcn_h(xl, e_ll, p[3], p[4], _N)
    gp = _gcn_h(xp, e_pp, p[5], p[6], _N)
    xl1 = _lr(gl)
    xp1 = _lr((sp + gp) / 2.0)
    sp2 = _sage_h(xl1, xp1, e_lp, p[7], p[8], p[9], _N)
    gl2 = _gcn_h(xl1, e_ll, p[10], p[11], _N)
    gp2 = _gcn_h(xp1, e_pp, p[12], p[13], _N)
    return _lr(gl2), _lr((sp2 + gp2) / 2.0)


def _final_body(hj_ref, hb_ref, x_ref, W_ref, b_ref, o_ref):
    acc = jnp.dot(x_ref[...], W_ref[...], preferred_element_type=jnp.float32)
    o_ref[...] = (hj_ref[...] + hb_ref[...]) * 0.5 + acc + b_ref[...]


def _final_combine(hj, hb, x, W, b):
    n, d = x.shape
    grid = (n // _BN,)
    return pl.pallas_call(
        _final_body,
        grid=grid,
        in_specs=[
            pl.BlockSpec((_BN, hj.shape[1]), lambda r: (r, 0)),
            pl.BlockSpec((_BN, hb.shape[1]), lambda r: (r, 0)),
            pl.BlockSpec((_BN, d), lambda r: (r, 0)),
            pl.BlockSpec((d, W.shape[1]), lambda r: (0, 0)),
            pl.BlockSpec((1, W.shape[1]), lambda r: (0, 0)),
        ],
        out_specs=pl.BlockSpec((_BN, W.shape[1]), lambda r: (r, 0)),
        out_shape=jax.ShapeDtypeStruct((n, W.shape[1]), jnp.float32),
    )(hj, hb, x, W, b.reshape(1, -1))


def kernel(x_lncRNA, x_protein, s1Wl_j, s1bl_j, s1Wr_j, g1lW_j, g1lb_j, g1pW_j, g1pb_j, s2Wl_j, s2bl_j, s2Wr_j, g2lW_j, g2lb_j, g2pW_j, g2pb_j, s1Wl_b, s1bl_b, s1Wr_b, g1lW_b, g1lb_b, g1pW_b, g1pb_b, s2Wl_b, s2bl_b, s2Wr_b, g2lW_b, g2lb_b, g2pW_b, g2pb_b, resL_W, resL_b, resP_W, resP_b, ei_lp_j, ei_ll_j, ei_pp_j, ei_lp_b, ei_ll_b, ei_pp_b):
    pj = (s1Wl_j, s1bl_j, s1Wr_j, g1lW_j, g1lb_j, g1pW_j, g1pb_j,
          s2Wl_j, s2bl_j, s2Wr_j, g2lW_j, g2lb_j, g2pW_j, g2pb_j)
    pb = (s1Wl_b, s1bl_b, s1Wr_b, g1lW_b, g1lb_b, g1pW_b, g1pb_b,
          s2Wl_b, s2bl_b, s2Wr_b, g2lW_b, g2lb_b, g2pW_b, g2pb_b)
    lj, pjp = _branch_h(x_lncRNA, x_protein, ei_lp_j, ei_ll_j, ei_pp_j, pj)
    lb, pbp = _branch_h(x_lncRNA, x_protein, ei_lp_b, ei_ll_b, ei_pp_b, pb)
    out_l = _final_combine(lj, lb, x_lncRNA, resL_W, resL_b)
    out_p = _final_combine(pjp, pbp, x_protein, resP_W, resP_b)
    return jnp.stack([out_l, out_p], axis=0)


# speedup vs baseline: 3.2259x; 3.2259x over previous
import functools

import jax
import jax.numpy as jnp
from jax import lax
from jax.experimental import pallas as pl
from jax.experimental.pallas import tpu as pltpu
from jax.experimental.pallas import tpu_sc as plsc

_N = 20000
_E = 320000
_NH = 10000
_PH = 10240
_PN = 20480
_EP = 327680
_NBAT = _EP // 128
_NBT = _NBAT // 16
_NBW = _NBAT // 32
_BN = 400
_RPB = _N // _BN // 2


def _mesh():
    return plsc.VectorSubcoreMesh(core_axis_name="c", subcore_axis_name="s")


def _prop_body(x2_hbm, idx3_hbm, dst3_hbm, out_hbm, acc, idxv, dstv, rows, zb,
               sem0, sem1):
    c = lax.axis_index("c")
    s = lax.axis_index("s")

    for i in range(8):
        for j in range(8):
            zb[i, pl.ds(j * 16, 16)] = jnp.zeros((16,), jnp.float32)
    for t in range(80):
        pltpu.sync_copy(zb, acc.at[pl.ds(s * 640 + t * 8, 8)])
    plsc.subcore_barrier()

    def _fire(j, half, sem):
        pltpu.async_copy(x2_hbm.at[idxv.at[j]], rows.at[half], sem)

    def _drain_scatter(j, half, sem):
        pltpu.make_async_copy(x2_hbm.at[idxv.at[j]], rows.at[half], sem).wait()
        pltpu.sync_copy(rows.at[half], acc.at[dstv.at[j]], add=True)

    def _chunk(k, carry):
        pltpu.sync_copy(idx3_hbm.at[pl.ds(s * _NBT + k * 16, 16)], idxv)
        pltpu.sync_copy(dst3_hbm.at[c, pl.ds(s * _NBT + k * 16, 16)], dstv)
        _fire(0, 0, sem0)

        def _step(t, cc):
            _fire(2 * t + 1, 1, sem1)
            _drain_scatter(2 * t, 0, sem0)

            @pl.when(t < 7)
            def _():
                _fire(2 * t + 2, 0, sem0)

            _drain_scatter(2 * t + 1, 1, sem1)
            return cc

        lax.fori_loop(0, 8, _step, carry)
        return carry

    lax.fori_loop(0, _NBT // 16, _chunk, 0)
    plsc.subcore_barrier()

    for t in range(5):
        pltpu.sync_copy(acc.at[pl.ds(s * 640 + t * 128, 128)], rows.at[0])
        pltpu.sync_copy(rows.at[0],
                        out_hbm.at[c, pl.ds(s * 640 + t * 128, 128)])


@functools.cache
def _prop_kernel():
    return pl.kernel(
        _prop_body,
        out_type=jax.ShapeDtypeStruct((2, _PH, 128), jnp.float32),
        mesh=_mesh(),
        scratch_types=[
            pltpu.VMEM_SHARED((_PH, 128), jnp.float32),
            pltpu.VMEM((16, 128), jnp.int32),
            pltpu.VMEM((16, 128), jnp.int32),
            pltpu.VMEM((2, 128, 128), jnp.float32),
            pltpu.VMEM((8, 128), jnp.float32),
            pltpu.SemaphoreType.DMA,
            pltpu.SemaphoreType.DMA,
        ],
    )


def _prop(x2, idx3, dst3):
    return _prop_kernel()(x2, idx3, dst3)


def _deg_body(dst3_hbm, out_hbm, acc, dstv, ones, zb, sem0):
    c = lax.axis_index("c")
    s = lax.axis_index("s")

    for j in range(80):
        zb[pl.ds(j * 16, 16)] = jnp.zeros((16,), jnp.float32)
    for j in range(8):
        ones[pl.ds(j * 16, 16)] = jnp.ones((16,), jnp.float32)
    pltpu.sync_copy(zb, acc.at[pl.ds(s * 1280, 1280)])
    plsc.subcore_barrier()

    w = c * 16 + s
    pltpu.sync_copy(dst3_hbm.at[pl.ds(w * _NBW, _NBW)], dstv)

    def _step(t, carry):
        for b in range(8):
            pltpu.async_copy(ones, acc.at[dstv.at[t * 8 + b]], sem0, add=True)
        for b in range(8):
            pltpu.make_async_copy(ones, acc.at[dstv.at[t * 8 + b]], sem0).wait()
        return carry

    lax.fori_loop(0, _NBW // 8, _step, 0)
    plsc.subcore_barrier()

    pltpu.sync_copy(acc.at[pl.ds(s * 1280, 1280)], zb)
    pltpu.sync_copy(zb, out_hbm.at[c, pl.ds(s * 1280, 1280)])


@functools.cache
def _degree_kernel():
    return pl.kernel(
        _deg_body,
        out_type=jax.ShapeDtypeStruct((2, _PN), jnp.float32),
        mesh=_mesh(),
        scratch_types=[
            pltpu.VMEM_SHARED((_PN,), jnp.float32),
            pltpu.VMEM((_NBW, 128), jnp.int32),
            pltpu.VMEM((128,), jnp.float32),
            pltpu.VMEM((1280,), jnp.float32),
            pltpu.SemaphoreType.DMA,
        ],
    )


def _degree(dst3):
    return _degree_kernel()(dst3)


def _lrelu(v):
    return jnp.where(v >= 0.0, v, 0.2 * v)


def _p_spec():
    return pl.BlockSpec((1, _BN, 128), lambda r: (r // _RPB, r % _RPB, 0))


def _scale_body(x_ref, s_ref, o_ref):
    o_ref[...] = x_ref[...] * s_ref[...]


def _scale(x, s):
    return pl.pallas_call(
        _scale_body,
        grid=(_N // _BN,),
        in_specs=[
            pl.BlockSpec((_BN, 128), lambda r: (r, 0)),
            pl.BlockSpec((_BN, 1), lambda r: (r, 0)),
        ],
        out_specs=pl.BlockSpec((_BN, 128), lambda r: (r, 0)),
        out_shape=jax.ShapeDtypeStruct((_N, 128), jnp.float32),
    )(x, s)


def _l1_body(xl_ref, xp_ref, Pll, Ppp, Plp,
             all_ref, app_ref, rml_ref,
             Wll_ref, Wpp_ref, Wsl_ref, Wr_ref,
             bl_ref, bp_ref, xl1_ref, xp1_ref):
    al = all_ref[...]
    ap = app_ref[...]
    rm = rml_ref[...]
    xl = xl_ref[...]
    xp = xp_ref[...]

    zl = (Pll[0] + xl * al) * al
    gl = jnp.dot(zl, Wll_ref[...], preferred_element_type=jnp.float32) + bl_ref[...]
    xl1_ref[...] = _lrelu(gl)

    zp = (Ppp[0] + xp * ap) * ap
    m = Plp[0] * rm
    accp = (jnp.dot(zp, Wpp_ref[...], preferred_element_type=jnp.float32)
            + jnp.dot(m, Wsl_ref[...], preferred_element_type=jnp.float32)
            + jnp.dot(xp, Wr_ref[...], preferred_element_type=jnp.float32)
            + bp_ref[...])
    xp1_ref[...] = _lrelu(0.5 * accp)


def _l1_fused(xl, xp, Pll, Ppp, Plp, a_ll, a_pp, rm_lp,
              g1lW, g1pW, s1Wl, s1Wr, b_l, b_p):
    col = pl.BlockSpec((_BN, 1), lambda r: (r, 0))
    wfull = pl.BlockSpec((128, 256), lambda r: (0, 0))
    bias = pl.BlockSpec((1, 256), lambda r: (0, 0))
    return pl.pallas_call(
        _l1_body,
        grid=(_N // _BN,),
        in_specs=[
            pl.BlockSpec((_BN, 128), lambda r: (r, 0)),
            pl.BlockSpec((_BN, 128), lambda r: (r, 0)),
            _p_spec(), _p_spec(), _p_spec(),
            col, col, col,
            wfull, wfull, wfull, wfull,
            bias, bias,
        ],
        out_specs=[
            pl.BlockSpec((_BN, 256), lambda r: (r, 0)),
            pl.BlockSpec((_BN, 256), lambda r: (r, 0)),
        ],
        out_shape=[
            jax.ShapeDtypeStruct((_N, 256), jnp.float32),
            jax.ShapeDtypeStruct((_N, 256), jnp.float32),
        ],
    )(xl, xp, Pll, Ppp, Plp, a_ll, a_pp, rm_lp,
      g1lW, g1pW, s1Wl, s1Wr, b_l.reshape(1, 256), b_p.reshape(1, 256))


def _l2_body(x_ref, W_ref, s_ref, o_ref):
    o_ref[0] = jnp.dot(x_ref[...], W_ref[0],
                       preferred_element_type=jnp.float32) * s_ref[0]


def _l2_matmul(x, W2, S2):
    return pl.pallas_call(
        _l2_body,
        grid=(_N // _BN, 2),
        in_specs=[
            pl.BlockSpec((_BN, 256), lambda r, g: (r, 0)),
            pl.BlockSpec((1, 256, 128), lambda r, g: (g, 0, 0)),
            pl.BlockSpec((1, _BN, 1), lambda r, g: (g, r, 0)),
        ],
        out_specs=pl.BlockSpec((1, _BN, 128), lambda r, g: (g, r, 0)),
        out_shape=jax.ShapeDtypeStruct((2, _N, 128), jnp.float32),
    )(x, W2, S2)


def _final_l_body(x_ref, W_ref, b_ref,
                  Pj, hsj, aj_ref, bj_ref,
                  Pb, hsb, ab_ref, bb_ref, o_ref):
    lj = _lrelu((Pj[0] + hsj[0]) * aj_ref[...] + bj_ref[...])
    lb = _lrelu((Pb[0] + hsb[0]) * ab_ref[...] + bb_ref[...])
    o_ref[...] = (0.5 * (lj + lb)
                  + jnp.dot(x_ref[...], W_ref[...],
                            preferred_element_type=jnp.float32)
                  + b_ref[...])


def _final_l(x, W, b, Pj, hcatj, aj, bj, Pb, hcatb, ab, bb):
    hs_spec = pl.BlockSpec((1, _BN, 128), lambda r: (0, r, 0))
    col = pl.BlockSpec((_BN, 1), lambda r: (r, 0))
    bias = pl.BlockSpec((1, 128), lambda r: (0, 0))
    return pl.pallas_call(
        _final_l_body,
        grid=(_N // _BN,),
        in_specs=[
            pl.BlockSpec((_BN, 128), lambda r: (r, 0)),
            pl.BlockSpec((128, 128), lambda r: (0, 0)),
            bias,
            _p_spec(), hs_spec, col, bias,
            _p_spec(), hs_spec, col, bias,
        ],
        out_specs=pl.BlockSpec((_BN, 128), lambda r: (r, 0)),
        out_shape=jax.ShapeDtypeStruct((_N, 128), jnp.float32),
    )(x, W, b.reshape(1, 128), Pj, hcatj, aj, bj.reshape(1, 128),
      Pb, hcatb, ab, bb.reshape(1, 128))


def _final_p_body(x_ref, W_ref, b_ref,
                  Plpj, rmj_ref, sblj_ref, dnj, Pppj, hspj, apj_ref, gbj_ref,
                  Plpb, rmb_ref, sblb_ref, dnb, Pppb, hspb, apb_ref, gbb_ref,
                  o_ref):
    sp2j = Plpj[0] * rmj_ref[...] + sblj_ref[...] + dnj[0]
    gp2j = (Pppj[0] + hspj[0]) * apj_ref[...] + gbj_ref[...]
    pj = _lrelu(0.5 * (sp2j + gp2j))
    sp2b = Plpb[0] * rmb_ref[...] + sblb_ref[...] + dnb[0]
    gp2b = (Pppb[0] + hspb[0]) * apb_ref[...] + gbb_ref[...]
    pb = _lrelu(0.5 * (sp2b + gp2b))
    o_ref[...] = (0.5 * (pj + pb)
                  + jnp.dot(x_ref[...], W_ref[...],
                            preferred_element_type=jnp.float32)
                  + b_ref[...])


def _final_p(x, W, b,
             Plpj, rmj, sblj, hcatj, Pppj, apj, gbj,
             Plpb, rmb, sblb, hcatb, Pppb, apb, gbb):
    hs_spec = pl.BlockSpec((1, _BN, 128), lambda r: (0, r, 0))
    dn_spec = pl.BlockSpec((1, _BN, 128), lambda r: (1, r, 0))
    col = pl.BlockSpec((_BN, 1), lambda r: (r, 0))
    bias = pl.BlockSpec((1, 128), lambda r: (0, 0))
    return pl.pallas_call(
        _final_p_body,
        grid=(_N // _BN,),
        in_specs=[
            pl.BlockSpec((_BN, 128), lambda r: (r, 0)),
            pl.BlockSpec((128, 128), lambda r: (0, 0)),
            bias,
            _p_spec(), col, bias, dn_spec, _p_spec(), hs_spec, col, bias,
            _p_spec(), col, bias, dn_spec, _p_spec(), hs_spec, col, bias,
        ],
        out_specs=pl.BlockSpec((_BN, 128), lambda r: (r, 0)),
        out_shape=jax.ShapeDtypeStruct((_N, 128), jnp.float32),
    )(x, W, b.reshape(1, 128),
      Plpj, rmj, sblj.reshape(1, 128), hcatj, Pppj, hcatj, apj, gbj.reshape(1, 128),
      Plpb, rmb, sblb.reshape(1, 128), hcatb, Pppb, hcatb, apb, gbb.reshape(1, 128))


def _prep_idx(src, base):
    pad = jnp.zeros((_EP - _E,), jnp.int32)
    return jnp.concatenate([src + base, pad]).reshape(_NBAT, 128)


def _prep_dst(dst):
    pad = jnp.full((_EP - _E,), 2 * _N, jnp.int32)
    dstp = jnp.concatenate([dst, pad])
    d0 = jnp.where(dstp < _NH, dstp, _NH)
    d1 = jnp.where((dstp >= _NH) & (dstp < _N), dstp - _NH, _NH)
    return jnp.stack([d0, d1]).reshape(2, _NBAT, 128)


def _prep_dst_deg(dst):
    pad = jnp.full((_EP - _E,), _N, jnp.int32)
    return jnp.concatenate([dst, pad]).reshape(_NBAT, 128)


def kernel(x_lncRNA, x_protein,
           s1Wl_j, s1bl_j, s1Wr_j, g1lW_j, g1lb_j, g1pW_j, g1pb_j,
           s2Wl_j, s2bl_j, s2Wr_j, g2lW_j, g2lb_j, g2pW_j, g2pb_j,
           s1Wl_b, s1bl_b, s1Wr_b, g1lW_b, g1lb_b, g1pW_b, g1pb_b,
           s2Wl_b, s2bl_b, s2Wr_b, g2lW_b, g2lb_b, g2pW_b, g2pb_b,
           resL_W, resL_b, resP_W, resP_b,
           ei_lp_j, ei_ll_j, ei_pp_j, ei_lp_b, ei_ll_b, ei_pp_b):
    xl, xp = x_lncRNA, x_protein
    ones_col = jnp.ones((_N, 1), jnp.float32)

    params = {
        'j': (s1Wl_j, s1bl_j, s1Wr_j, g1lW_j, g1lb_j, g1pW_j, g1pb_j,
              s2Wl_j, s2bl_j, s2Wr_j, g2lW_j, g2lb_j, g2pW_j, g2pb_j),
        'b': (s1Wl_b, s1bl_b, s1Wr_b, g1lW_b, g1lb_b, g1pW_b, g1pb_b,
              s2Wl_b, s2bl_b, s2Wr_b, g2lW_b, g2lb_b, g2pW_b, g2pb_b),
    }
    edges = {'j': (ei_lp_j, ei_ll_j, ei_pp_j), 'b': (ei_lp_b, ei_ll_b, ei_pp_b)}

    res = {}
    for q in ('j', 'b'):
        (s1Wl, s1bl, s1Wr, g1lW, g1lb, g1pW, g1pb,
         s2Wl, s2bl, s2Wr, g2lW, g2lb, g2pW, g2pb) = params[q]
        e_lp, e_ll, e_pp = edges[q]

        dst_lp = _prep_dst(e_lp[1])
        dst_ll = _prep_dst(e_ll[1])
        dst_pp = _prep_dst(e_pp[1])
        idx_lp = _prep_idx(e_lp[0], 0)
        idx_ll = _prep_idx(e_ll[0], 0)
        idx_pp = _prep_idx(e_pp[0], 0)
        idx_lp2 = _prep_idx(e_lp[0], _N)

        dll = _degree(_prep_dst_deg(e_ll[1]))
        dpp = _degree(_prep_dst_deg(e_pp[1]))
        clp = _degree(_prep_dst_deg(e_lp[1]))
        a_ll = lax.rsqrt(dll[0, :_N] + dll[1, :_N] + 1.0)[:, None]
        a_pp = lax.rsqrt(dpp[0, :_N] + dpp[1, :_N] + 1.0)[:, None]
        rm_lp = (1.0 / jnp.maximum(clp[0, :_N] + clp[1, :_N], 1.0))[:, None]

        xls = _scale(xl, a_ll)
        xps = _scale(xp, a_pp)
        Pll = _prop(xls, idx_ll, dst_ll)
        Ppp = _prop(xps, idx_pp, dst_pp)
        Plp = _prop(xl, idx_lp, dst_lp)

        xl1, xp1 = _l1_fused(xl, xp, Pll, Ppp, Plp, a_ll, a_pp, rm_lp,
                             g1lW, g1pW, s1Wl, s1Wr,
                             g1lb, s1bl + g1pb)

        W2_l = jnp.stack([g2lW, s2Wl])
        W2_p = jnp.stack([g2pW, s2Wr])
        S_l = jnp.stack([a_ll, ones_col])
        S_p = jnp.stack([a_pp, ones_col])
        hcat_l = _l2_matmul(xl1, W2_l, S_l)
        hcat_p = _l2_matmul(xp1, W2_p, S_p)

        Pll2 = _prop(hcat_l.reshape(2 * _N, 128), idx_ll, dst_ll)
        Plp2 = _prop(hcat_l.reshape(2 * _N, 128), idx_lp2, dst_lp)
        Ppp2 = _prop(hcat_p.reshape(2 * _N, 128), idx_pp, dst_pp)

        res[q] = (Pll2, hcat_l, a_ll, g2lb,
                  Plp2, rm_lp, s2bl, hcat_p, Ppp2, a_pp, g2pb)

    (Pll2j, hclj, allj, g2lbj, Plp2j, rmlpj, s2blj, hcpj, Ppp2j, appj, g2pbj) = res['j']
    (Pll2b, hclb, allb, g2lbb, Plp2b, rmlpb, s2blb, hcpb, Ppp2b, appb, g2pbb) = res['b']

    out_l = _final_l(xl, resL_W, resL_b,
                     Pll2j, hclj, allj, g2lbj,
                     Pll2b, hclb, allb, g2lbb)
    out_p = _final_p(xp, resP_W, resP_b,
                     Plp2j, rmlpj, s2blj, hcpj, Ppp2j, appj, g2pbj,
                     Plp2b, rmlpb, s2blb, hcpb, Ppp2b, appb, g2pbb)
    return jnp.stack([out_l, out_p], axis=0)


# speedup vs baseline: 3.3530x; 1.0394x over previous
import functools

import jax
import jax.numpy as jnp
from jax import lax
from jax.experimental import pallas as pl
from jax.experimental.pallas import tpu as pltpu
from jax.experimental.pallas import tpu_sc as plsc

_N = 20000
_E = 320000
_NH = 10000
_PH = 10240
_PN = 20480
_EP = 327680
_NBAT = _EP // 128
_NBT = _NBAT // 16
_NBW = _NBAT // 32
_BN = 400
_RPB = _N // _BN // 2


def _mesh():
    return plsc.VectorSubcoreMesh(core_axis_name="c", subcore_axis_name="s")


def _prop_body(x2_hbm, idx3_hbm, dst3_hbm, out_hbm, acc, idxv, dstv, rows, zb,
               sem0, sem1):
    c = lax.axis_index("c")
    s = lax.axis_index("s")

    for i in range(8):
        for j in range(8):
            zb[i, pl.ds(j * 16, 16)] = jnp.zeros((16,), jnp.float32)
    for t in range(80):
        pltpu.sync_copy(zb, acc.at[pl.ds(s * 640 + t * 8, 8)])
    plsc.subcore_barrier()

    def _fire(j, half, sem):
        pltpu.async_copy(x2_hbm.at[idxv.at[j]], rows.at[half], sem)

    def _drain_scatter(j, half, sem):
        pltpu.make_async_copy(x2_hbm.at[idxv.at[j]], rows.at[half], sem).wait()
        pltpu.sync_copy(rows.at[half], acc.at[dstv.at[j]], add=True)

    def _chunk(k, carry):
        pltpu.sync_copy(idx3_hbm.at[pl.ds(s * _NBT + k * 16, 16)], idxv)
        pltpu.sync_copy(dst3_hbm.at[c, pl.ds(s * _NBT + k * 16, 16)], dstv)
        _fire(0, 0, sem0)

        def _step(t, cc):
            _fire(2 * t + 1, 1, sem1)
            _drain_scatter(2 * t, 0, sem0)

            @pl.when(t < 7)
            def _():
                _fire(2 * t + 2, 0, sem0)

            _drain_scatter(2 * t + 1, 1, sem1)
            return cc

        lax.fori_loop(0, 8, _step, carry)
        return carry

    lax.fori_loop(0, _NBT // 16, _chunk, 0)
    plsc.subcore_barrier()

    for t in range(5):
        pltpu.sync_copy(acc.at[pl.ds(s * 640 + t * 128, 128)], rows.at[0])
        pltpu.sync_copy(rows.at[0],
                        out_hbm.at[c, pl.ds(s * 640 + t * 128, 128)])


@functools.cache
def _prop_kernel():
    return pl.kernel(
        _prop_body,
        out_type=jax.ShapeDtypeStruct((2, _PH, 128), jnp.float32),
        mesh=_mesh(),
        scratch_types=[
            pltpu.VMEM_SHARED((_PH, 128), jnp.float32),
            pltpu.VMEM((16, 128), jnp.int32),
            pltpu.VMEM((16, 128), jnp.int32),
            pltpu.VMEM((2, 128, 128), jnp.float32),
            pltpu.VMEM((8, 128), jnp.float32),
            pltpu.SemaphoreType.DMA,
            pltpu.SemaphoreType.DMA,
        ],
    )


def _prop(x2, idx3, dst3):
    return _prop_kernel()(x2, idx3, dst3)


def _deg_body(dst3_hbm, out_hbm, acc, dstv, ones, zb, sem0):
    c = lax.axis_index("c")
    s = lax.axis_index("s")

    for j in range(80):
        zb[pl.ds(j * 16, 16)] = jnp.zeros((16,), jnp.float32)
    for j in range(8):
        ones[pl.ds(j * 16, 16)] = jnp.ones((16,), jnp.float32)
    pltpu.sync_copy(zb, acc.at[pl.ds(s * 1280, 1280)])
    plsc.subcore_barrier()

    w = c * 16 + s
    pltpu.sync_copy(dst3_hbm.at[pl.ds(w * _NBW, _NBW)], dstv)

    def _step(t, carry):
        for b in range(8):
            pltpu.async_copy(ones, acc.at[dstv.at[t * 8 + b]], sem0, add=True)
        for b in range(8):
            pltpu.make_async_copy(ones, acc.at[dstv.at[t * 8 + b]], sem0).wait()
        return carry

    lax.fori_loop(0, _NBW // 8, _step, 0)
    plsc.subcore_barrier()

    pltpu.sync_copy(acc.at[pl.ds(s * 1280, 1280)], zb)
    pltpu.sync_copy(zb, out_hbm.at[c, pl.ds(s * 1280, 1280)])


@functools.cache
def _degree_kernel():
    return pl.kernel(
        _deg_body,
        out_type=jax.ShapeDtypeStruct((2, _PN), jnp.float32),
        mesh=_mesh(),
        scratch_types=[
            pltpu.VMEM_SHARED((_PN,), jnp.float32),
            pltpu.VMEM((_NBW, 128), jnp.int32),
            pltpu.VMEM((128,), jnp.float32),
            pltpu.VMEM((1280,), jnp.float32),
            pltpu.SemaphoreType.DMA,
        ],
    )


def _degree(dst3):
    return _degree_kernel()(dst3)


def _lrelu(v):
    return jnp.where(v >= 0.0, v, 0.2 * v)


def _p_spec():
    return pl.BlockSpec((1, _BN, 128), lambda r: (r // _RPB, r % _RPB, 0))


def _scale_body(x_ref, s_ref, o_ref):
    o_ref[...] = x_ref[...] * s_ref[...]


def _scale(x, s):
    return pl.pallas_call(
        _scale_body,
        grid=(_N // _BN,),
        in_specs=[
            pl.BlockSpec((_BN, 128), lambda r: (r, 0)),
            pl.BlockSpec((_BN, 1), lambda r: (r, 0)),
        ],
        out_specs=pl.BlockSpec((_BN, 128), lambda r: (r, 0)),
        out_shape=jax.ShapeDtypeStruct((_N, 128), jnp.float32),
    )(x, s)


def _l1_body(xl_ref, xp_ref, Pll, Ppp, Plp,
             all_ref, app_ref, rml_ref,
             Wll_ref, Wpp_ref, Wsl_ref, Wr_ref,
             bl_ref, bp_ref, xl1_ref, xp1_ref):
    al = all_ref[...]
    ap = app_ref[...]
    rm = rml_ref[...]
    xl = xl_ref[...]
    xp = xp_ref[...]

    zl = (Pll[0] + xl * al) * al
    gl = jnp.dot(zl, Wll_ref[...], preferred_element_type=jnp.float32) + bl_ref[...]
    xl1_ref[...] = _lrelu(gl)

    zp = (Ppp[0] + xp * ap) * ap
    m = Plp[0] * rm
    accp = (jnp.dot(zp, Wpp_ref[...], preferred_element_type=jnp.float32)
            + jnp.dot(m, Wsl_ref[...], preferred_element_type=jnp.float32)
            + jnp.dot(xp, Wr_ref[...], preferred_element_type=jnp.float32)
            + bp_ref[...])
    xp1_ref[...] = _lrelu(0.5 * accp)


def _l1_fused(xl, xp, Pll, Ppp, Plp, a_ll, a_pp, rm_lp,
              g1lW, g1pW, s1Wl, s1Wr, b_l, b_p):
    col = pl.BlockSpec((_BN, 1), lambda r: (r, 0))
    wfull = pl.BlockSpec((128, 256), lambda r: (0, 0))
    bias = pl.BlockSpec((1, 256), lambda r: (0, 0))
    return pl.pallas_call(
        _l1_body,
        grid=(_N // _BN,),
        in_specs=[
            pl.BlockSpec((_BN, 128), lambda r: (r, 0)),
            pl.BlockSpec((_BN, 128), lambda r: (r, 0)),
            _p_spec(), _p_spec(), _p_spec(),
            col, col, col,
            wfull, wfull, wfull, wfull,
            bias, bias,
        ],
        out_specs=[
            pl.BlockSpec((_BN, 256), lambda r: (r, 0)),
            pl.BlockSpec((_BN, 256), lambda r: (r, 0)),
        ],
        out_shape=[
            jax.ShapeDtypeStruct((_N, 256), jnp.float32),
            jax.ShapeDtypeStruct((_N, 256), jnp.float32),
        ],
    )(xl, xp, Pll, Ppp, Plp, a_ll, a_pp, rm_lp,
      g1lW, g1pW, s1Wl, s1Wr, b_l.reshape(1, 256), b_p.reshape(1, 256))


def _l2_body(x_ref, W_ref, s_ref, o_ref):
    o_ref[0] = jnp.dot(x_ref[...], W_ref[0],
                       preferred_element_type=jnp.float32) * s_ref[0]


def _l2_matmul(x, W2, S2):
    return pl.pallas_call(
        _l2_body,
        grid=(_N // _BN, 2),
        in_specs=[
            pl.BlockSpec((_BN, 256), lambda r, g: (r, 0)),
            pl.BlockSpec((1, 256, 128), lambda r, g: (g, 0, 0)),
            pl.BlockSpec((1, _BN, 1), lambda r, g: (g, r, 0)),
        ],
        out_specs=pl.BlockSpec((1, _BN, 128), lambda r, g: (g, r, 0)),
        out_shape=jax.ShapeDtypeStruct((2, _N, 128), jnp.float32),
    )(x, W2, S2)


def _final_l_body(x_ref, W_ref, b_ref,
                  Pj, hsj, aj_ref, bj_ref,
                  Pb, hsb, ab_ref, bb_ref, o_ref):
    lj = _lrelu((Pj[0] + hsj[0]) * aj_ref[...] + bj_ref[...])
    lb = _lrelu((Pb[0] + hsb[0]) * ab_ref[...] + bb_ref[...])
    o_ref[...] = (0.5 * (lj + lb)
                  + jnp.dot(x_ref[...], W_ref[...],
                            preferred_element_type=jnp.float32)
                  + b_ref[...])


def _final_l(x, W, b, Pj, hcatj, aj, bj, Pb, hcatb, ab, bb):
    hs_spec = pl.BlockSpec((1, _BN, 128), lambda r: (0, r, 0))
    col = pl.BlockSpec((_BN, 1), lambda r: (r, 0))
    bias = pl.BlockSpec((1, 128), lambda r: (0, 0))
    return pl.pallas_call(
        _final_l_body,
        grid=(_N // _BN,),
        in_specs=[
            pl.BlockSpec((_BN, 128), lambda r: (r, 0)),
            pl.BlockSpec((128, 128), lambda r: (0, 0)),
            bias,
            _p_spec(), hs_spec, col, bias,
            _p_spec(), hs_spec, col, bias,
        ],
        out_specs=pl.BlockSpec((_BN, 128), lambda r: (r, 0)),
        out_shape=jax.ShapeDtypeStruct((_N, 128), jnp.float32),
    )(x, W, b.reshape(1, 128), Pj, hcatj, aj, bj.reshape(1, 128),
      Pb, hcatb, ab, bb.reshape(1, 128))


def _final_p_body(x_ref, W_ref, b_ref,
                  Plpj, rmj_ref, sblj_ref, dnj, Pppj, hspj, apj_ref, gbj_ref,
                  Plpb, rmb_ref, sblb_ref, dnb, Pppb, hspb, apb_ref, gbb_ref,
                  o_ref):
    sp2j = Plpj[0] * rmj_ref[...] + sblj_ref[...] + dnj[0]
    gp2j = (Pppj[0] + hspj[0]) * apj_ref[...] + gbj_ref[...]
    pj = _lrelu(0.5 * (sp2j + gp2j))
    sp2b = Plpb[0] * rmb_ref[...] + sblb_ref[...] + dnb[0]
    gp2b = (Pppb[0] + hspb[0]) * apb_ref[...] + gbb_ref[...]
    pb = _lrelu(0.5 * (sp2b + gp2b))
    o_ref[...] = (0.5 * (pj + pb)
                  + jnp.dot(x_ref[...], W_ref[...],
                            preferred_element_type=jnp.float32)
                  + b_ref[...])


def _final_p(x, W, b,
             Plpj, rmj, sblj, hcatj, Pppj, apj, gbj,
             Plpb, rmb, sblb, hcatb, Pppb, apb, gbb):
    hs_spec = pl.BlockSpec((1, _BN, 128), lambda r: (0, r, 0))
    dn_spec = pl.BlockSpec((1, _BN, 128), lambda r: (1, r, 0))
    col = pl.BlockSpec((_BN, 1), lambda r: (r, 0))
    bias = pl.BlockSpec((1, 128), lambda r: (0, 0))
    return pl.pallas_call(
        _final_p_body,
        grid=(_N // _BN,),
        in_specs=[
            pl.BlockSpec((_BN, 128), lambda r: (r, 0)),
            pl.BlockSpec((128, 128), lambda r: (0, 0)),
            bias,
            _p_spec(), col, bias, dn_spec, _p_spec(), hs_spec, col, bias,
            _p_spec(), col, bias, dn_spec, _p_spec(), hs_spec, col, bias,
        ],
        out_specs=pl.BlockSpec((_BN, 128), lambda r: (r, 0)),
        out_shape=jax.ShapeDtypeStruct((_N, 128), jnp.float32),
    )(x, W, b.reshape(1, 128),
      Plpj, rmj, sblj.reshape(1, 128), hcatj, Pppj, hcatj, apj, gbj.reshape(1, 128),
      Plpb, rmb, sblb.reshape(1, 128), hcatb, Pppb, hcatb, apb, gbb.reshape(1, 128))


def _prep_idx(src, base):
    pad = jnp.zeros((_EP - _E,), jnp.int32)
    return jnp.concatenate([src + base, pad]).reshape(_NBAT, 128)


def _prep_dst(dst):
    pad = jnp.full((_EP - _E,), 2 * _N, jnp.int32)
    dstp = jnp.concatenate([dst, pad])
    dump = _NH + jnp.arange(_EP, dtype=jnp.int32) % (_PH - _NH)
    d0 = jnp.where(dstp < _NH, dstp, dump)
    d1 = jnp.where((dstp >= _NH) & (dstp < _N), dstp - _NH, dump)
    return jnp.stack([d0, d1]).reshape(2, _NBAT, 128)


def _prep_dst_deg(dst):
    pad = jnp.full((_EP - _E,), _N, jnp.int32)
    return jnp.concatenate([dst, pad]).reshape(_NBAT, 128)


def kernel(x_lncRNA, x_protein,
           s1Wl_j, s1bl_j, s1Wr_j, g1lW_j, g1lb_j, g1pW_j, g1pb_j,
           s2Wl_j, s2bl_j, s2Wr_j, g2lW_j, g2lb_j, g2pW_j, g2pb_j,
           s1Wl_b, s1bl_b, s1Wr_b, g1lW_b, g1lb_b, g1pW_b, g1pb_b,
           s2Wl_b, s2bl_b, s2Wr_b, g2lW_b, g2lb_b, g2pW_b, g2pb_b,
           resL_W, resL_b, resP_W, resP_b,
           ei_lp_j, ei_ll_j, ei_pp_j, ei_lp_b, ei_ll_b, ei_pp_b):
    xl, xp = x_lncRNA, x_protein
    ones_col = jnp.ones((_N, 1), jnp.float32)

    params = {
        'j': (s1Wl_j, s1bl_j, s1Wr_j, g1lW_j, g1lb_j, g1pW_j, g1pb_j,
              s2Wl_j, s2bl_j, s2Wr_j, g2lW_j, g2lb_j, g2pW_j, g2pb_j),
        'b': (s1Wl_b, s1bl_b, s1Wr_b, g1lW_b, g1lb_b, g1pW_b, g1pb_b,
              s2Wl_b, s2bl_b, s2Wr_b, g2lW_b, g2lb_b, g2pW_b, g2pb_b),
    }
    edges = {'j': (ei_lp_j, ei_ll_j, ei_pp_j), 'b': (ei_lp_b, ei_ll_b, ei_pp_b)}

    res = {}
    for q in ('j', 'b'):
        (s1Wl, s1bl, s1Wr, g1lW, g1lb, g1pW, g1pb,
         s2Wl, s2bl, s2Wr, g2lW, g2lb, g2pW, g2pb) = params[q]
        e_lp, e_ll, e_pp = edges[q]

        dst_lp = _prep_dst(e_lp[1])
        dst_ll = _prep_dst(e_ll[1])
        dst_pp = _prep_dst(e_pp[1])
        idx_lp = _prep_idx(e_lp[0], 0)
        idx_ll = _prep_idx(e_ll[0], 0)
        idx_pp = _prep_idx(e_pp[0], 0)
        idx_lp2 = _prep_idx(e_lp[0], _N)

        dll = _degree(_prep_dst_deg(e_ll[1]))
        dpp = _degree(_prep_dst_deg(e_pp[1]))
        clp = _degree(_prep_dst_deg(e_lp[1]))
        a_ll = lax.rsqrt(dll[0, :_N] + dll[1, :_N] + 1.0)[:, None]
        a_pp = lax.rsqrt(dpp[0, :_N] + dpp[1, :_N] + 1.0)[:, None]
        rm_lp = (1.0 / jnp.maximum(clp[0, :_N] + clp[1, :_N], 1.0))[:, None]

        xls = _scale(xl, a_ll)
        xps = _scale(xp, a_pp)
        Pll = _prop(xls, idx_ll, dst_ll)
        Ppp = _prop(xps, idx_pp, dst_pp)
        Plp = _prop(xl, idx_lp, dst_lp)

        xl1, xp1 = _l1_fused(xl, xp, Pll, Ppp, Plp, a_ll, a_pp, rm_lp,
                             g1lW, g1pW, s1Wl, s1Wr,
                             g1lb, s1bl + g1pb)

        W2_l = jnp.stack([g2lW, s2Wl])
        W2_p = jnp.stack([g2pW, s2Wr])
        S_l = jnp.stack([a_ll, ones_col])
        S_p = jnp.stack([a_pp, ones_col])
        hcat_l = _l2_matmul(xl1, W2_l, S_l)
        hcat_p = _l2_matmul(xp1, W2_p, S_p)

        Pll2 = _prop(hcat_l.reshape(2 * _N, 128), idx_ll, dst_ll)
        Plp2 = _prop(hcat_l.reshape(2 * _N, 128), idx_lp2, dst_lp)
        Ppp2 = _prop(hcat_p.reshape(2 * _N, 128), idx_pp, dst_pp)

        res[q] = (Pll2, hcat_l, a_ll, g2lb,
                  Plp2, rm_lp, s2bl, hcat_p, Ppp2, a_pp, g2pb)

    (Pll2j, hclj, allj, g2lbj, Plp2j, rmlpj, s2blj, hcpj, Ppp2j, appj, g2pbj) = res['j']
    (Pll2b, hclb, allb, g2lbb, Plp2b, rmlpb, s2blb, hcpb, Ppp2b, appb, g2pbb) = res['b']

    out_l = _final_l(xl, resL_W, resL_b,
                     Pll2j, hclj, allj, g2lbj,
                     Pll2b, hclb, allb, g2lbb)
    out_p = _final_p(xp, resP_W, resP_b,
                     Plp2j, rmlpj, s2blj, hcpj, Ppp2j, appj, g2pbj,
                     Plp2b, rmlpb, s2blb, hcpb, Ppp2b, appb, g2pbb)
    return jnp.stack([out_l, out_p], axis=0)


# speedup vs baseline: 3.3557x; 1.0008x over previous
import functools

import jax
import jax.numpy as jnp
from jax import lax
from jax.experimental import pallas as pl
from jax.experimental.pallas import tpu as pltpu
from jax.experimental.pallas import tpu_sc as plsc

_N = 20000
_E = 320000
_NH = 10000
_PH = 10240
_PN = 20480
_EP = 327680
_NBAT = _EP // 128
_NBT = _NBAT // 16
_NBW = _NBAT // 32
_BN = 400
_RPB = _N // _BN // 2


def _mesh():
    return plsc.VectorSubcoreMesh(core_axis_name="c", subcore_axis_name="s")


def _prop_body(x2_hbm, idx3_hbm, dst3_hbm, out_hbm, acc, idxv, dstv, rows, zb,
               sem0, sem1):
    c = lax.axis_index("c")
    s = lax.axis_index("s")

    for i in range(8):
        for j in range(8):
            zb[i, pl.ds(j * 16, 16)] = jnp.zeros((16,), jnp.float32)
    for t in range(80):
        pltpu.sync_copy(zb, acc.at[pl.ds(s * 640 + t * 8, 8)])
    plsc.subcore_barrier()

    def _fire(j, half, sem):
        for p in range(4):
            pltpu.async_copy(x2_hbm.at[idxv.at[j, pl.ds(p * 32, 32)]],
                             rows.at[half, pl.ds(p * 32, 32)], sem)

    def _drain_scatter(j, half, sem):
        for p in range(4):
            pltpu.make_async_copy(x2_hbm.at[idxv.at[j, pl.ds(p * 32, 32)]],
                                  rows.at[half, pl.ds(p * 32, 32)], sem).wait()
        pltpu.sync_copy(rows.at[half], acc.at[dstv.at[j]], add=True)

    def _chunk(k, carry):
        pltpu.sync_copy(idx3_hbm.at[pl.ds(s * _NBT + k * 16, 16)], idxv)
        pltpu.sync_copy(dst3_hbm.at[c, pl.ds(s * _NBT + k * 16, 16)], dstv)
        _fire(0, 0, sem0)

        def _step(t, cc):
            _fire(2 * t + 1, 1, sem1)
            _drain_scatter(2 * t, 0, sem0)

            @pl.when(t < 7)
            def _():
                _fire(2 * t + 2, 0, sem0)

            _drain_scatter(2 * t + 1, 1, sem1)
            return cc

        lax.fori_loop(0, 8, _step, carry)
        return carry

    lax.fori_loop(0, _NBT // 16, _chunk, 0)
    plsc.subcore_barrier()

    for t in range(5):
        pltpu.sync_copy(acc.at[pl.ds(s * 640 + t * 128, 128)], rows.at[0])
        pltpu.sync_copy(rows.at[0],
                        out_hbm.at[c, pl.ds(s * 640 + t * 128, 128)])


@functools.cache
def _prop_kernel():
    return pl.kernel(
        _prop_body,
        out_type=jax.ShapeDtypeStruct((2, _PH, 128), jnp.float32),
        mesh=_mesh(),
        scratch_types=[
            pltpu.VMEM_SHARED((_PH, 128), jnp.float32),
            pltpu.VMEM((16, 128), jnp.int32),
            pltpu.VMEM((16, 128), jnp.int32),
            pltpu.VMEM((2, 128, 128), jnp.float32),
            pltpu.VMEM((8, 128), jnp.float32),
            pltpu.SemaphoreType.DMA,
            pltpu.SemaphoreType.DMA,
        ],
    )


def _prop(x2, idx3, dst3):
    return _prop_kernel()(x2, idx3, dst3)


def _deg_body(dst3_hbm, out_hbm, acc, dstv, ones, zb, sem0):
    c = lax.axis_index("c")
    s = lax.axis_index("s")

    for j in range(80):
        zb[pl.ds(j * 16, 16)] = jnp.zeros((16,), jnp.float32)
    for j in range(8):
        ones[pl.ds(j * 16, 16)] = jnp.ones((16,), jnp.float32)
    pltpu.sync_copy(zb, acc.at[pl.ds(s * 1280, 1280)])
    plsc.subcore_barrier()

    w = c * 16 + s
    pltpu.sync_copy(dst3_hbm.at[pl.ds(w * _NBW, _NBW)], dstv)

    def _step(t, carry):
        for b in range(8):
            pltpu.async_copy(ones, acc.at[dstv.at[t * 8 + b]], sem0, add=True)
        for b in range(8):
            pltpu.make_async_copy(ones, acc.at[dstv.at[t * 8 + b]], sem0).wait()
        return carry

    lax.fori_loop(0, _NBW // 8, _step, 0)
    plsc.subcore_barrier()

    pltpu.sync_copy(acc.at[pl.ds(s * 1280, 1280)], zb)
    pltpu.sync_copy(zb, out_hbm.at[c, pl.ds(s * 1280, 1280)])


@functools.cache
def _degree_kernel():
    return pl.kernel(
        _deg_body,
        out_type=jax.ShapeDtypeStruct((2, _PN), jnp.float32),
        mesh=_mesh(),
        scratch_types=[
            pltpu.VMEM_SHARED((_PN,), jnp.float32),
            pltpu.VMEM((_NBW, 128), jnp.int32),
            pltpu.VMEM((128,), jnp.float32),
            pltpu.VMEM((1280,), jnp.float32),
            pltpu.SemaphoreType.DMA,
        ],
    )


def _degree(dst3):
    return _degree_kernel()(dst3)


def _lrelu(v):
    return jnp.where(v >= 0.0, v, 0.2 * v)


def _p_spec():
    return pl.BlockSpec((1, _BN, 128), lambda r: (r // _RPB, r % _RPB, 0))


def _scale_body(x_ref, s_ref, o_ref):
    o_ref[...] = x_ref[...] * s_ref[...]


def _scale(x, s):
    return pl.pallas_call(
        _scale_body,
        grid=(_N // _BN,),
        in_specs=[
            pl.BlockSpec((_BN, 128), lambda r: (r, 0)),
            pl.BlockSpec((_BN, 1), lambda r: (r, 0)),
        ],
        out_specs=pl.BlockSpec((_BN, 128), lambda r: (r, 0)),
        out_shape=jax.ShapeDtypeStruct((_N, 128), jnp.float32),
    )(x, s)


def _l1_body(xl_ref, xp_ref, Pll, Ppp, Plp,
             all_ref, app_ref, rml_ref,
             Wll_ref, Wpp_ref, Wsl_ref, Wr_ref,
             bl_ref, bp_ref, xl1_ref, xp1_ref):
    al = all_ref[...]
    ap = app_ref[...]
    rm = rml_ref[...]
    xl = xl_ref[...]
    xp = xp_ref[...]

    zl = (Pll[0] + xl * al) * al
    gl = jnp.dot(zl, Wll_ref[...], preferred_element_type=jnp.float32) + bl_ref[...]
    xl1_ref[...] = _lrelu(gl)

    zp = (Ppp[0] + xp * ap) * ap
    m = Plp[0] * rm
    accp = (jnp.dot(zp, Wpp_ref[...], preferred_element_type=jnp.float32)
            + jnp.dot(m, Wsl_ref[...], preferred_element_type=jnp.float32)
            + jnp.dot(xp, Wr_ref[...], preferred_element_type=jnp.float32)
            + bp_ref[...])
    xp1_ref[...] = _lrelu(0.5 * accp)


def _l1_fused(xl, xp, Pll, Ppp, Plp, a_ll, a_pp, rm_lp,
              g1lW, g1pW, s1Wl, s1Wr, b_l, b_p):
    col = pl.BlockSpec((_BN, 1), lambda r: (r, 0))
    wfull = pl.BlockSpec((128, 256), lambda r: (0, 0))
    bias = pl.BlockSpec((1, 256), lambda r: (0, 0))
    return pl.pallas_call(
        _l1_body,
        grid=(_N // _BN,),
        in_specs=[
            pl.BlockSpec((_BN, 128), lambda r: (r, 0)),
            pl.BlockSpec((_BN, 128), lambda r: (r, 0)),
            _p_spec(), _p_spec(), _p_spec(),
            col, col, col,
            wfull, wfull, wfull, wfull,
            bias, bias,
        ],
        out_specs=[
            pl.BlockSpec((_BN, 256), lambda r: (r, 0)),
            pl.BlockSpec((_BN, 256), lambda r: (r, 0)),
        ],
        out_shape=[
            jax.ShapeDtypeStruct((_N, 256), jnp.float32),
            jax.ShapeDtypeStruct((_N, 256), jnp.float32),
        ],
    )(xl, xp, Pll, Ppp, Plp, a_ll, a_pp, rm_lp,
      g1lW, g1pW, s1Wl, s1Wr, b_l.reshape(1, 256), b_p.reshape(1, 256))


def _l2_body(x_ref, W_ref, s_ref, o_ref):
    o_ref[0] = jnp.dot(x_ref[...], W_ref[0],
                       preferred_element_type=jnp.float32) * s_ref[0]


def _l2_matmul(x, W2, S2):
    return pl.pallas_call(
        _l2_body,
        grid=(_N // _BN, 2),
        in_specs=[
            pl.BlockSpec((_BN, 256), lambda r, g: (r, 0)),
            pl.BlockSpec((1, 256, 128), lambda r, g: (g, 0, 0)),
            pl.BlockSpec((1, _BN, 1), lambda r, g: (g, r, 0)),
        ],
        out_specs=pl.BlockSpec((1, _BN, 128), lambda r, g: (g, r, 0)),
        out_shape=jax.ShapeDtypeStruct((2, _N, 128), jnp.float32),
    )(x, W2, S2)


def _final_l_body(x_ref, W_ref, b_ref,
                  Pj, hsj, aj_ref, bj_ref,
                  Pb, hsb, ab_ref, bb_ref, o_ref):
    lj = _lrelu((Pj[0] + hsj[0]) * aj_ref[...] + bj_ref[...])
    lb = _lrelu((Pb[0] + hsb[0]) * ab_ref[...] + bb_ref[...])
    o_ref[...] = (0.5 * (lj + lb)
                  + jnp.dot(x_ref[...], W_ref[...],
                            preferred_element_type=jnp.float32)
                  + b_ref[...])


def _final_l(x, W, b, Pj, hcatj, aj, bj, Pb, hcatb, ab, bb):
    hs_spec = pl.BlockSpec((1, _BN, 128), lambda r: (0, r, 0))
    col = pl.BlockSpec((_BN, 1), lambda r: (r, 0))
    bias = pl.BlockSpec((1, 128), lambda r: (0, 0))
    return pl.pallas_call(
        _final_l_body,
        grid=(_N // _BN,),
        in_specs=[
            pl.BlockSpec((_BN, 128), lambda r: (r, 0)),
            pl.BlockSpec((128, 128), lambda r: (0, 0)),
            bias,
            _p_spec(), hs_spec, col, bias,
            _p_spec(), hs_spec, col, bias,
        ],
        out_specs=pl.BlockSpec((_BN, 128), lambda r: (r, 0)),
        out_shape=jax.ShapeDtypeStruct((_N, 128), jnp.float32),
    )(x, W, b.reshape(1, 128), Pj, hcatj, aj, bj.reshape(1, 128),
      Pb, hcatb, ab, bb.reshape(1, 128))


def _final_p_body(x_ref, W_ref, b_ref,
                  Plpj, rmj_ref, sblj_ref, dnj, Pppj, hspj, apj_ref, gbj_ref,
                  Plpb, rmb_ref, sblb_ref, dnb, Pppb, hspb, apb_ref, gbb_ref,
                  o_ref):
    sp2j = Plpj[0] * rmj_ref[...] + sblj_ref[...] + dnj[0]
    gp2j = (Pppj[0] + hspj[0]) * apj_ref[...] + gbj_ref[...]
    pj = _lrelu(0.5 * (sp2j + gp2j))
    sp2b = Plpb[0] * rmb_ref[...] + sblb_ref[...] + dnb[0]
    gp2b = (Pppb[0] + hspb[0]) * apb_ref[...] + gbb_ref[...]
    pb = _lrelu(0.5 * (sp2b + gp2b))
    o_ref[...] = (0.5 * (pj + pb)
                  + jnp.dot(x_ref[...], W_ref[...],
                            preferred_element_type=jnp.float32)
                  + b_ref[...])


def _final_p(x, W, b,
             Plpj, rmj, sblj, hcatj, Pppj, apj, gbj,
             Plpb, rmb, sblb, hcatb, Pppb, apb, gbb):
    hs_spec = pl.BlockSpec((1, _BN, 128), lambda r: (0, r, 0))
    dn_spec = pl.BlockSpec((1, _BN, 128), lambda r: (1, r, 0))
    col = pl.BlockSpec((_BN, 1), lambda r: (r, 0))
    bias = pl.BlockSpec((1, 128), lambda r: (0, 0))
    return pl.pallas_call(
        _final_p_body,
        grid=(_N // _BN,),
        in_specs=[
            pl.BlockSpec((_BN, 128), lambda r: (r, 0)),
            pl.BlockSpec((128, 128), lambda r: (0, 0)),
            bias,
            _p_spec(), col, bias, dn_spec, _p_spec(), hs_spec, col, bias,
            _p_spec(), col, bias, dn_spec, _p_spec(), hs_spec, col, bias,
        ],
        out_specs=pl.BlockSpec((_BN, 128), lambda r: (r, 0)),
        out_shape=jax.ShapeDtypeStruct((_N, 128), jnp.float32),
    )(x, W, b.reshape(1, 128),
      Plpj, rmj, sblj.reshape(1, 128), hcatj, Pppj, hcatj, apj, gbj.reshape(1, 128),
      Plpb, rmb, sblb.reshape(1, 128), hcatb, Pppb, hcatb, apb, gbb.reshape(1, 128))


def _prep_idx(src, base):
    pad = jnp.zeros((_EP - _E,), jnp.int32)
    return jnp.concatenate([src + base, pad]).reshape(_NBAT, 128)


def _prep_dst(dst):
    pad = jnp.full((_EP - _E,), 2 * _N, jnp.int32)
    dstp = jnp.concatenate([dst, pad])
    dump = _NH + jnp.arange(_EP, dtype=jnp.int32) % (_PH - _NH)
    d0 = jnp.where(dstp < _NH, dstp, dump)
    d1 = jnp.where((dstp >= _NH) & (dstp < _N), dstp - _NH, dump)
    return jnp.stack([d0, d1]).reshape(2, _NBAT, 128)


def _prep_dst_deg(dst):
    pad = jnp.full((_EP - _E,), _N, jnp.int32)
    return jnp.concatenate([dst, pad]).reshape(_NBAT, 128)


def kernel(x_lncRNA, x_protein,
           s1Wl_j, s1bl_j, s1Wr_j, g1lW_j, g1lb_j, g1pW_j, g1pb_j,
           s2Wl_j, s2bl_j, s2Wr_j, g2lW_j, g2lb_j, g2pW_j, g2pb_j,
           s1Wl_b, s1bl_b, s1Wr_b, g1lW_b, g1lb_b, g1pW_b, g1pb_b,
           s2Wl_b, s2bl_b, s2Wr_b, g2lW_b, g2lb_b, g2pW_b, g2pb_b,
           resL_W, resL_b, resP_W, resP_b,
           ei_lp_j, ei_ll_j, ei_pp_j, ei_lp_b, ei_ll_b, ei_pp_b):
    xl, xp = x_lncRNA, x_protein
    ones_col = jnp.ones((_N, 1), jnp.float32)

    params = {
        'j': (s1Wl_j, s1bl_j, s1Wr_j, g1lW_j, g1lb_j, g1pW_j, g1pb_j,
              s2Wl_j, s2bl_j, s2Wr_j, g2lW_j, g2lb_j, g2pW_j, g2pb_j),
        'b': (s1Wl_b, s1bl_b, s1Wr_b, g1lW_b, g1lb_b, g1pW_b, g1pb_b,
              s2Wl_b, s2bl_b, s2Wr_b, g2lW_b, g2lb_b, g2pW_b, g2pb_b),
    }
    edges = {'j': (ei_lp_j, ei_ll_j, ei_pp_j), 'b': (ei_lp_b, ei_ll_b, ei_pp_b)}

    res = {}
    for q in ('j', 'b'):
        (s1Wl, s1bl, s1Wr, g1lW, g1lb, g1pW, g1pb,
         s2Wl, s2bl, s2Wr, g2lW, g2lb, g2pW, g2pb) = params[q]
        e_lp, e_ll, e_pp = edges[q]

        dst_lp = _prep_dst(e_lp[1])
        dst_ll = _prep_dst(e_ll[1])
        dst_pp = _prep_dst(e_pp[1])
        idx_lp = _prep_idx(e_lp[0], 0)
        idx_ll = _prep_idx(e_ll[0], 0)
        idx_pp = _prep_idx(e_pp[0], 0)
        idx_lp2 = _prep_idx(e_lp[0], _N)

        dll = _degree(_prep_dst_deg(e_ll[1]))
        dpp = _degree(_prep_dst_deg(e_pp[1]))
        clp = _degree(_prep_dst_deg(e_lp[1]))
        a_ll = lax.rsqrt(dll[0, :_N] + dll[1, :_N] + 1.0)[:, None]
        a_pp = lax.rsqrt(dpp[0, :_N] + dpp[1, :_N] + 1.0)[:, None]
        rm_lp = (1.0 / jnp.maximum(clp[0, :_N] + clp[1, :_N], 1.0))[:, None]

        xls = _scale(xl, a_ll)
        xps = _scale(xp, a_pp)
        Pll = _prop(xls, idx_ll, dst_ll)
        Ppp = _prop(xps, idx_pp, dst_pp)
        Plp = _prop(xl, idx_lp, dst_lp)

        xl1, xp1 = _l1_fused(xl, xp, Pll, Ppp, Plp, a_ll, a_pp, rm_lp,
                             g1lW, g1pW, s1Wl, s1Wr,
                             g1lb, s1bl + g1pb)

        W2_l = jnp.stack([g2lW, s2Wl])
        W2_p = jnp.stack([g2pW, s2Wr])
        S_l = jnp.stack([a_ll, ones_col])
        S_p = jnp.stack([a_pp, ones_col])
        hcat_l = _l2_matmul(xl1, W2_l, S_l)
        hcat_p = _l2_matmul(xp1, W2_p, S_p)

        Pll2 = _prop(hcat_l.reshape(2 * _N, 128), idx_ll, dst_ll)
        Plp2 = _prop(hcat_l.reshape(2 * _N, 128), idx_lp2, dst_lp)
        Ppp2 = _prop(hcat_p.reshape(2 * _N, 128), idx_pp, dst_pp)

        res[q] = (Pll2, hcat_l, a_ll, g2lb,
                  Plp2, rm_lp, s2bl, hcat_p, Ppp2, a_pp, g2pb)

    (Pll2j, hclj, allj, g2lbj, Plp2j, rmlpj, s2blj, hcpj, Ppp2j, appj, g2pbj) = res['j']
    (Pll2b, hclb, allb, g2lbb, Plp2b, rmlpb, s2blb, hcpb, Ppp2b, appb, g2pbb) = res['b']

    out_l = _final_l(xl, resL_W, resL_b,
                     Pll2j, hclj, allj, g2lbj,
                     Pll2b, hclb, allb, g2lbb)
    out_p = _final_p(xp, resP_W, resP_b,
                     Plp2j, rmlpj, s2blj, hcpj, Ppp2j, appj, g2pbj,
                     Plp2b, rmlpb, s2blb, hcpb, Ppp2b, appb, g2pbb)
    return jnp.stack([out_l, out_p], axis=0)


# speedup vs baseline: 3.3680x; 1.0037x over previous
import functools

import jax
import jax.numpy as jnp
from jax import lax
from jax.experimental import pallas as pl
from jax.experimental.pallas import tpu as pltpu
from jax.experimental.pallas import tpu_sc as plsc

_N = 20000
_E = 320000
_NH = 10000
_PH = 10240
_PN = 20480
_EP = 327680
_NBAT = _EP // 128
_NBT = _NBAT // 16
_NBW = _NBAT // 32
_BN = 400
_RPB = _N // _BN // 2


def _mesh():
    return plsc.VectorSubcoreMesh(core_axis_name="c", subcore_axis_name="s")


def _prop_body(x2_hbm, idx3_hbm, dst3_hbm, out_hbm, acc, idxv, dstv, rows, zb,
               sem0, sem1):
    c = lax.axis_index("c")
    s = lax.axis_index("s")

    for i in range(32):
        for j in range(8):
            zb[i, pl.ds(j * 16, 16)] = jnp.zeros((16,), jnp.float32)
    for t in range(20):
        pltpu.sync_copy(zb, acc.at[pl.ds(s * 640 + t * 32, 32)])
    plsc.subcore_barrier()

    def _fire(j, half, sem):
        pltpu.async_copy(x2_hbm.at[idxv.at[j]], rows.at[half], sem)

    def _drain_scatter(j, half, sem):
        pltpu.make_async_copy(x2_hbm.at[idxv.at[j]], rows.at[half], sem).wait()
        pltpu.sync_copy(rows.at[half], acc.at[dstv.at[j]], add=True)

    def _chunk(k, carry):
        pltpu.sync_copy(idx3_hbm.at[pl.ds(s * _NBT + k * 16, 16)], idxv)
        pltpu.sync_copy(dst3_hbm.at[c, pl.ds(s * _NBT + k * 16, 16)], dstv)
        _fire(0, 0, sem0)

        def _step(t, cc):
            _fire(2 * t + 1, 1, sem1)
            _drain_scatter(2 * t, 0, sem0)

            @pl.when(t < 7)
            def _():
                _fire(2 * t + 2, 0, sem0)

            _drain_scatter(2 * t + 1, 1, sem1)
            return cc

        lax.fori_loop(0, 8, _step, carry)
        return carry

    lax.fori_loop(0, _NBT // 16, _chunk, 0)
    plsc.subcore_barrier()

    pltpu.async_copy(acc.at[pl.ds(s * 640, 128)], rows.at[0], sem0)
    for t in range(5):
        pltpu.make_async_copy(acc.at[pl.ds(s * 640 + t * 128, 128)],
                              rows.at[t % 2], sem0).wait()
        if t > 0:
            pltpu.make_async_copy(rows.at[(t - 1) % 2],
                                  out_hbm.at[c, pl.ds(s * 640 + (t - 1) * 128, 128)],
                                  sem1).wait()
        if t < 4:
            pltpu.async_copy(acc.at[pl.ds(s * 640 + (t + 1) * 128, 128)],
                             rows.at[(t + 1) % 2], sem0)
        pltpu.async_copy(rows.at[t % 2],
                         out_hbm.at[c, pl.ds(s * 640 + t * 128, 128)], sem1)
    pltpu.make_async_copy(rows.at[0],
                          out_hbm.at[c, pl.ds(s * 640 + 4 * 128, 128)],
                          sem1).wait()


@functools.cache
def _prop_kernel():
    return pl.kernel(
        _prop_body,
        out_type=jax.ShapeDtypeStruct((2, _PH, 128), jnp.float32),
        mesh=_mesh(),
        scratch_types=[
            pltpu.VMEM_SHARED((_PH, 128), jnp.float32),
            pltpu.VMEM((16, 128), jnp.int32),
            pltpu.VMEM((16, 128), jnp.int32),
            pltpu.VMEM((2, 128, 128), jnp.float32),
            pltpu.VMEM((32, 128), jnp.float32),
            pltpu.SemaphoreType.DMA,
            pltpu.SemaphoreType.DMA,
        ],
    )


def _prop(x2, idx3, dst3):
    return _prop_kernel()(x2, idx3, dst3)


def _deg_body(dst3_hbm, out_hbm, acc, dstv, ones, zb, sem0):
    c = lax.axis_index("c")
    s = lax.axis_index("s")

    for j in range(80):
        zb[pl.ds(j * 16, 16)] = jnp.zeros((16,), jnp.float32)
    for j in range(8):
        ones[pl.ds(j * 16, 16)] = jnp.ones((16,), jnp.float32)
    pltpu.sync_copy(zb, acc.at[pl.ds(s * 1280, 1280)])
    plsc.subcore_barrier()

    w = c * 16 + s
    pltpu.sync_copy(dst3_hbm.at[pl.ds(w * _NBW, _NBW)], dstv)

    def _step(t, carry):
        for b in range(8):
            pltpu.async_copy(ones, acc.at[dstv.at[t * 8 + b]], sem0, add=True)
        for b in range(8):
            pltpu.make_async_copy(ones, acc.at[dstv.at[t * 8 + b]], sem0).wait()
        return carry

    lax.fori_loop(0, _NBW // 8, _step, 0)
    plsc.subcore_barrier()

    pltpu.sync_copy(acc.at[pl.ds(s * 1280, 1280)], zb)
    pltpu.sync_copy(zb, out_hbm.at[c, pl.ds(s * 1280, 1280)])


@functools.cache
def _degree_kernel():
    return pl.kernel(
        _deg_body,
        out_type=jax.ShapeDtypeStruct((2, _PN), jnp.float32),
        mesh=_mesh(),
        scratch_types=[
            pltpu.VMEM_SHARED((_PN,), jnp.float32),
            pltpu.VMEM((_NBW, 128), jnp.int32),
            pltpu.VMEM((128,), jnp.float32),
            pltpu.VMEM((1280,), jnp.float32),
            pltpu.SemaphoreType.DMA,
        ],
    )


def _degree(dst3):
    return _degree_kernel()(dst3)


def _lrelu(v):
    return jnp.where(v >= 0.0, v, 0.2 * v)


def _p_spec():
    return pl.BlockSpec((1, _BN, 128), lambda r: (r // _RPB, r % _RPB, 0))


def _scale_body(x_ref, s_ref, o_ref):
    o_ref[...] = x_ref[...] * s_ref[...]


def _scale(x, s):
    return pl.pallas_call(
        _scale_body,
        grid=(_N // _BN,),
        in_specs=[
            pl.BlockSpec((_BN, 128), lambda r: (r, 0)),
            pl.BlockSpec((_BN, 1), lambda r: (r, 0)),
        ],
        out_specs=pl.BlockSpec((_BN, 128), lambda r: (r, 0)),
        out_shape=jax.ShapeDtypeStruct((_N, 128), jnp.float32),
    )(x, s)


def _l1_body(xl_ref, xp_ref, Pll, Ppp, Plp,
             all_ref, app_ref, rml_ref,
             Wll_ref, Wpp_ref, Wsl_ref, Wr_ref,
             bl_ref, bp_ref, xl1_ref, xp1_ref):
    al = all_ref[...]
    ap = app_ref[...]
    rm = rml_ref[...]
    xl = xl_ref[...]
    xp = xp_ref[...]

    zl = (Pll[0] + xl * al) * al
    gl = jnp.dot(zl, Wll_ref[...], preferred_element_type=jnp.float32) + bl_ref[...]
    xl1_ref[...] = _lrelu(gl)

    zp = (Ppp[0] + xp * ap) * ap
    m = Plp[0] * rm
    accp = (jnp.dot(zp, Wpp_ref[...], preferred_element_type=jnp.float32)
            + jnp.dot(m, Wsl_ref[...], preferred_element_type=jnp.float32)
            + jnp.dot(xp, Wr_ref[...], preferred_element_type=jnp.float32)
            + bp_ref[...])
    xp1_ref[...] = _lrelu(0.5 * accp)


def _l1_fused(xl, xp, Pll, Ppp, Plp, a_ll, a_pp, rm_lp,
              g1lW, g1pW, s1Wl, s1Wr, b_l, b_p):
    col = pl.BlockSpec((_BN, 1), lambda r: (r, 0))
    wfull = pl.BlockSpec((128, 256), lambda r: (0, 0))
    bias = pl.BlockSpec((1, 256), lambda r: (0, 0))
    return pl.pallas_call(
        _l1_body,
        grid=(_N // _BN,),
        in_specs=[
            pl.BlockSpec((_BN, 128), lambda r: (r, 0)),
            pl.BlockSpec((_BN, 128), lambda r: (r, 0)),
            _p_spec(), _p_spec(), _p_spec(),
            col, col, col,
            wfull, wfull, wfull, wfull,
            bias, bias,
        ],
        out_specs=[
            pl.BlockSpec((_BN, 256), lambda r: (r, 0)),
            pl.BlockSpec((_BN, 256), lambda r: (r, 0)),
        ],
        out_shape=[
            jax.ShapeDtypeStruct((_N, 256), jnp.float32),
            jax.ShapeDtypeStruct((_N, 256), jnp.float32),
        ],
    )(xl, xp, Pll, Ppp, Plp, a_ll, a_pp, rm_lp,
      g1lW, g1pW, s1Wl, s1Wr, b_l.reshape(1, 256), b_p.reshape(1, 256))


def _l2_body(x_ref, W_ref, s_ref, o_ref):
    o_ref[0] = jnp.dot(x_ref[...], W_ref[0],
                       preferred_element_type=jnp.float32) * s_ref[0]


def _l2_matmul(x, W2, S2):
    return pl.pallas_call(
        _l2_body,
        grid=(_N // _BN, 2),
        in_specs=[
            pl.BlockSpec((_BN, 256), lambda r, g: (r, 0)),
            pl.BlockSpec((1, 256, 128), lambda r, g: (g, 0, 0)),
            pl.BlockSpec((1, _BN, 1), lambda r, g: (g, r, 0)),
        ],
        out_specs=pl.BlockSpec((1, _BN, 128), lambda r, g: (g, r, 0)),
        out_shape=jax.ShapeDtypeStruct((2, _N, 128), jnp.float32),
    )(x, W2, S2)


def _final_l_body(x_ref, W_ref, b_ref,
                  Pj, hsj, aj_ref, bj_ref,
                  Pb, hsb, ab_ref, bb_ref, o_ref):
    lj = _lrelu((Pj[0] + hsj[0]) * aj_ref[...] + bj_ref[...])
    lb = _lrelu((Pb[0] + hsb[0]) * ab_ref[...] + bb_ref[...])
    o_ref[...] = (0.5 * (lj + lb)
                  + jnp.dot(x_ref[...], W_ref[...],
                            preferred_element_type=jnp.float32)
                  + b_ref[...])


def _final_l(x, W, b, Pj, hcatj, aj, bj, Pb, hcatb, ab, bb):
    hs_spec = pl.BlockSpec((1, _BN, 128), lambda r: (0, r, 0))
    col = pl.BlockSpec((_BN, 1), lambda r: (r, 0))
    bias = pl.BlockSpec((1, 128), lambda r: (0, 0))
    return pl.pallas_call(
        _final_l_body,
        grid=(_N // _BN,),
        in_specs=[
            pl.BlockSpec((_BN, 128), lambda r: (r, 0)),
            pl.BlockSpec((128, 128), lambda r: (0, 0)),
            bias,
            _p_spec(), hs_spec, col, bias,
            _p_spec(), hs_spec, col, bias,
        ],
        out_specs=pl.BlockSpec((_BN, 128), lambda r: (r, 0)),
        out_shape=jax.ShapeDtypeStruct((_N, 128), jnp.float32),
    )(x, W, b.reshape(1, 128), Pj, hcatj, aj, bj.reshape(1, 128),
      Pb, hcatb, ab, bb.reshape(1, 128))


def _final_p_body(x_ref, W_ref, b_ref,
                  Plpj, rmj_ref, sblj_ref, dnj, Pppj, hspj, apj_ref, gbj_ref,
                  Plpb, rmb_ref, sblb_ref, dnb, Pppb, hspb, apb_ref, gbb_ref,
                  o_ref):
    sp2j = Plpj[0] * rmj_ref[...] + sblj_ref[...] + dnj[0]
    gp2j = (Pppj[0] + hspj[0]) * apj_ref[...] + gbj_ref[...]
    pj = _lrelu(0.5 * (sp2j + gp2j))
    sp2b = Plpb[0] * rmb_ref[...] + sblb_ref[...] + dnb[0]
    gp2b = (Pppb[0] + hspb[0]) * apb_ref[...] + gbb_ref[...]
    pb = _lrelu(0.5 * (sp2b + gp2b))
    o_ref[...] = (0.5 * (pj + pb)
                  + jnp.dot(x_ref[...], W_ref[...],
                            preferred_element_type=jnp.float32)
                  + b_ref[...])


def _final_p(x, W, b,
             Plpj, rmj, sblj, hcatj, Pppj, apj, gbj,
             Plpb, rmb, sblb, hcatb, Pppb, apb, gbb):
    hs_spec = pl.BlockSpec((1, _BN, 128), lambda r: (0, r, 0))
    dn_spec = pl.BlockSpec((1, _BN, 128), lambda r: (1, r, 0))
    col = pl.BlockSpec((_BN, 1), lambda r: (r, 0))
    bias = pl.BlockSpec((1, 128), lambda r: (0, 0))
    return pl.pallas_call(
        _final_p_body,
        grid=(_N // _BN,),
        in_specs=[
            pl.BlockSpec((_BN, 128), lambda r: (r, 0)),
            pl.BlockSpec((128, 128), lambda r: (0, 0)),
            bias,
            _p_spec(), col, bias, dn_spec, _p_spec(), hs_spec, col, bias,
            _p_spec(), col, bias, dn_spec, _p_spec(), hs_spec, col, bias,
        ],
        out_specs=pl.BlockSpec((_BN, 128), lambda r: (r, 0)),
        out_shape=jax.ShapeDtypeStruct((_N, 128), jnp.float32),
    )(x, W, b.reshape(1, 128),
      Plpj, rmj, sblj.reshape(1, 128), hcatj, Pppj, hcatj, apj, gbj.reshape(1, 128),
      Plpb, rmb, sblb.reshape(1, 128), hcatb, Pppb, hcatb, apb, gbb.reshape(1, 128))


def _prep_idx(src, base):
    pad = jnp.zeros((_EP - _E,), jnp.int32)
    return jnp.concatenate([src + base, pad]).reshape(_NBAT, 128)


def _prep_dst(dst):
    pad = jnp.full((_EP - _E,), 2 * _N, jnp.int32)
    dstp = jnp.concatenate([dst, pad])
    dump = _NH + jnp.arange(_EP, dtype=jnp.int32) % (_PH - _NH)
    d0 = jnp.where(dstp < _NH, dstp, dump)
    d1 = jnp.where((dstp >= _NH) & (dstp < _N), dstp - _NH, dump)
    return jnp.stack([d0, d1]).reshape(2, _NBAT, 128)


def _prep_dst_deg(dst):
    pad = jnp.full((_EP - _E,), _N, jnp.int32)
    return jnp.concatenate([dst, pad]).reshape(_NBAT, 128)


def kernel(x_lncRNA, x_protein,
           s1Wl_j, s1bl_j, s1Wr_j, g1lW_j, g1lb_j, g1pW_j, g1pb_j,
           s2Wl_j, s2bl_j, s2Wr_j, g2lW_j, g2lb_j, g2pW_j, g2pb_j,
           s1Wl_b, s1bl_b, s1Wr_b, g1lW_b, g1lb_b, g1pW_b, g1pb_b,
           s2Wl_b, s2bl_b, s2Wr_b, g2lW_b, g2lb_b, g2pW_b, g2pb_b,
           resL_W, resL_b, resP_W, resP_b,
           ei_lp_j, ei_ll_j, ei_pp_j, ei_lp_b, ei_ll_b, ei_pp_b):
    xl, xp = x_lncRNA, x_protein
    ones_col = jnp.ones((_N, 1), jnp.float32)

    params = {
        'j': (s1Wl_j, s1bl_j, s1Wr_j, g1lW_j, g1lb_j, g1pW_j, g1pb_j,
              s2Wl_j, s2bl_j, s2Wr_j, g2lW_j, g2lb_j, g2pW_j, g2pb_j),
        'b': (s1Wl_b, s1bl_b, s1Wr_b, g1lW_b, g1lb_b, g1pW_b, g1pb_b,
              s2Wl_b, s2bl_b, s2Wr_b, g2lW_b, g2lb_b, g2pW_b, g2pb_b),
    }
    edges = {'j': (ei_lp_j, ei_ll_j, ei_pp_j), 'b': (ei_lp_b, ei_ll_b, ei_pp_b)}

    res = {}
    for q in ('j', 'b'):
        (s1Wl, s1bl, s1Wr, g1lW, g1lb, g1pW, g1pb,
         s2Wl, s2bl, s2Wr, g2lW, g2lb, g2pW, g2pb) = params[q]
        e_lp, e_ll, e_pp = edges[q]

        dst_lp = _prep_dst(e_lp[1])
        dst_ll = _prep_dst(e_ll[1])
        dst_pp = _prep_dst(e_pp[1])
        idx_lp = _prep_idx(e_lp[0], 0)
        idx_ll = _prep_idx(e_ll[0], 0)
        idx_pp = _prep_idx(e_pp[0], 0)
        idx_lp2 = _prep_idx(e_lp[0], _N)

        dll = _degree(_prep_dst_deg(e_ll[1]))
        dpp = _degree(_prep_dst_deg(e_pp[1]))
        clp = _degree(_prep_dst_deg(e_lp[1]))
        a_ll = lax.rsqrt(dll[0, :_N] + dll[1, :_N] + 1.0)[:, None]
        a_pp = lax.rsqrt(dpp[0, :_N] + dpp[1, :_N] + 1.0)[:, None]
        rm_lp = (1.0 / jnp.maximum(clp[0, :_N] + clp[1, :_N], 1.0))[:, None]

        xls = _scale(xl, a_ll)
        xps = _scale(xp, a_pp)
        Pll = _prop(xls, idx_ll, dst_ll)
        Ppp = _prop(xps, idx_pp, dst_pp)
        Plp = _prop(xl, idx_lp, dst_lp)

        xl1, xp1 = _l1_fused(xl, xp, Pll, Ppp, Plp, a_ll, a_pp, rm_lp,
                             g1lW, g1pW, s1Wl, s1Wr,
                             g1lb, s1bl + g1pb)

        W2_l = jnp.stack([g2lW, s2Wl])
        W2_p = jnp.stack([g2pW, s2Wr])
        S_l = jnp.stack([a_ll, ones_col])
        S_p = jnp.stack([a_pp, ones_col])
        hcat_l = _l2_matmul(xl1, W2_l, S_l)
        hcat_p = _l2_matmul(xp1, W2_p, S_p)

        Pll2 = _prop(hcat_l.reshape(2 * _N, 128), idx_ll, dst_ll)
        Plp2 = _prop(hcat_l.reshape(2 * _N, 128), idx_lp2, dst_lp)
        Ppp2 = _prop(hcat_p.reshape(2 * _N, 128), idx_pp, dst_pp)

        res[q] = (Pll2, hcat_l, a_ll, g2lb,
                  Plp2, rm_lp, s2bl, hcat_p, Ppp2, a_pp, g2pb)

    (Pll2j, hclj, allj, g2lbj, Plp2j, rmlpj, s2blj, hcpj, Ppp2j, appj, g2pbj) = res['j']
    (Pll2b, hclb, allb, g2lbb, Plp2b, rmlpb, s2blb, hcpb, Ppp2b, appb, g2pbb) = res['b']

    out_l = _final_l(xl, resL_W, resL_b,
                     Pll2j, hclj, allj, g2lbj,
                     Pll2b, hclb, allb, g2lbb)
    out_p = _final_p(xp, resP_W, resP_b,
                     Plp2j, rmlpj, s2blj, hcpj, Ppp2j, appj, g2pbj,
                     Plp2b, rmlpb, s2blb, hcpb, Ppp2b, appb, g2pbb)
    return jnp.stack([out_l, out_p], axis=0)
